# baseline (speedup 1.0000x reference)
import functools

import jax
import jax.numpy as jnp
from jax import lax
from jax.experimental import pallas as pl
from jax.experimental.pallas import tpu as pltpu
from jax.experimental.pallas import tpu_sc as plsc


_BS = 2048


def _add_kernel(x_ref, emb_ref, out_ref):
    out_ref[...] = x_ref[...] + emb_ref[...]


def _kernel_tc(x, emb_table, nbatch=None):
    batch, seq, d = x.shape
    if nbatch is None:
        nbatch = batch
    pos = emb_table[:seq]
    bs = _BS if seq % _BS == 0 else seq
    grid = (seq // bs, nbatch)
    return pl.pallas_call(
        _add_kernel,
        grid=grid,
        in_specs=[
            pl.BlockSpec((1, bs, d), lambda i, j: (j, i, 0)),
            pl.BlockSpec((bs, d), lambda i, j: (i, 0)),
        ],
        out_specs=pl.BlockSpec((1, bs, d), lambda i, j: (j, i, 0)),
        out_shape=jax.ShapeDtypeStruct((nbatch, seq, d), x.dtype),
    )(x, pos)


_CROWS = 8
_UNROLL = 4
_NG = 3


def _make_sc(batch, seq, d, batch_off=0):
    info = plsc.get_sparse_core_info()
    nw = info.num_cores * info.num_subcores
    lpw = seq // nw
    n_chunks = lpw // _CROWS
    cw = _CROWS * d
    mesh = plsc.VectorSubcoreMesh(core_axis_name="c", subcore_axis_name="s")

    scratch = (
        [pltpu.VMEM((cw,), jnp.float32) for _ in range(_NG)]
        + [pltpu.VMEM((cw,), jnp.float32) for _ in range(_NG * batch)]
        + [pltpu.SemaphoreType.DMA for _ in range(_NG * (1 + 2 * batch))]
    )

    @functools.partial(
        pl.kernel,
        mesh=mesh,
        out_type=jax.ShapeDtypeStruct((batch * seq * d,), jnp.float32),
        scratch_types=scratch,
    )
    def k(x_hbm, emb_hbm, out_hbm, *scr):
        ebuf = scr[:_NG]
        xbuf = scr[_NG:_NG + _NG * batch]
        sems = scr[_NG + _NG * batch:]
        esem = sems[:_NG]
        xlsem = sems[_NG:_NG + _NG * batch]
        xssem = sems[_NG + _NG * batch:]
        wid = lax.axis_index("s") * info.num_cores + lax.axis_index("c")
        descs = {}

        def fire_loads(t):
            g = t % _NG
            lstart = (wid * lpw + t * _CROWS) * d
            descs[("e", t)] = pltpu.async_copy(
                emb_hbm.at[pl.ds(lstart, cw)], ebuf[g], esem[g])
            for b in range(batch):
                src = x_hbm.at[pl.ds((batch_off + b) * seq * d + lstart, cw)]
                descs[("x", t, b)] = pltpu.async_copy(
                    src, xbuf[g * batch + b], xlsem[g * batch + b])

        fire_loads(0)
        if n_chunks > 1:
            fire_loads(1)
        for t in range(n_chunks):
            g = t % _NG
            descs[("e", t)].wait()
            for b in range(batch):
                descs[("x", t, b)].wait()
            eb = ebuf[g]
            xbs = [xbuf[g * batch + b] for b in range(batch)]

            def body(i, _, eb=eb, xbs=xbs):
                for u in range(_UNROLL):
                    off = (i * _UNROLL + u) * 16
                    e = eb[pl.ds(off, 16)]
                    for xb in xbs:
                        plsc.addupdate(xb.at[pl.ds(off, 16)], e)
                return 0

            lax.fori_loop(0, cw // (16 * _UNROLL), body, 0)
            lstart = (wid * lpw + t * _CROWS) * d
            for b in range(batch):
                dst = out_hbm.at[pl.ds(b * seq * d + lstart, cw)]
                descs[("s", t, b)] = pltpu.async_copy(
                    xbuf[g * batch + b], dst, xssem[g * batch + b])
            if t + 2 < n_chunks:
                if t >= 1:
                    for b in range(batch):
                        descs[("s", t - 1, b)].wait()
                fire_loads(t + 2)
        for t in range(max(0, n_chunks - 3), n_chunks):
            for b in range(batch):
                if ("s", t, b) in descs:
                    descs[("s", t, b)].wait()

    return k


def _kernel_sc(x, emb_table):
    batch, seq, d = x.shape
    pos = emb_table[:seq]
    out = _make_sc(batch, seq, d)(x.reshape(-1), pos.reshape(-1))
    return out.reshape(batch, seq, d)


def _kernel_hybrid(x, emb_table, sc_batches=1):
    batch, seq, d = x.shape
    pos = emb_table[:seq]
    tc_b = batch - sc_batches
    tc_out = _kernel_tc(x, emb_table, nbatch=tc_b)
    sc_out = _make_sc(sc_batches, seq, d, batch_off=tc_b)(
        x.reshape(-1), pos.reshape(-1))
    return jnp.concatenate([tc_out, sc_out.reshape(sc_batches, seq, d)], axis=0)


def kernel(x, emb_table):
    return _kernel_sc(x, emb_table)


# speedup vs baseline: 1.0261x; 1.0261x over previous
import functools

import jax
import jax.numpy as jnp
from jax import lax
from jax.experimental import pallas as pl
from jax.experimental.pallas import tpu as pltpu
from jax.experimental.pallas import tpu_sc as plsc


_BS = 2048


def _add_kernel(x_ref, emb_ref, out_ref):
    out_ref[...] = x_ref[...] + emb_ref[...]


def _kernel_tc(x, emb_table, nbatch=None):
    batch, seq, d = x.shape
    if nbatch is None:
        nbatch = batch
    pos = emb_table[:seq]
    bs = _BS if seq % _BS == 0 else seq
    grid = (seq // bs, nbatch)
    return pl.pallas_call(
        _add_kernel,
        grid=grid,
        in_specs=[
            pl.BlockSpec((1, bs, d), lambda i, j: (j, i, 0)),
            pl.BlockSpec((bs, d), lambda i, j: (i, 0)),
        ],
        out_specs=pl.BlockSpec((1, bs, d), lambda i, j: (j, i, 0)),
        out_shape=jax.ShapeDtypeStruct((nbatch, seq, d), x.dtype),
    )(x, pos)


_CROWS = 8
_UNROLL = 4
_NG = 3


def _make_sc(batch, seq, d, batch_off=0):
    info = plsc.get_sparse_core_info()
    nw = info.num_cores * info.num_subcores
    lpw = seq // nw
    n_chunks = lpw // _CROWS
    cw = _CROWS * d
    mesh = plsc.VectorSubcoreMesh(core_axis_name="c", subcore_axis_name="s")

    scratch = (
        [pltpu.VMEM((cw,), jnp.float32) for _ in range(_NG)]
        + [pltpu.VMEM((cw,), jnp.float32) for _ in range(_NG * batch)]
        + [pltpu.SemaphoreType.DMA for _ in range(_NG * (1 + 2 * batch))]
    )

    @functools.partial(
        pl.kernel,
        mesh=mesh,
        out_type=jax.ShapeDtypeStruct((batch * seq * d,), jnp.float32),
        scratch_types=scratch,
    )
    def k(x_hbm, emb_hbm, out_hbm, *scr):
        ebuf = scr[:_NG]
        xbuf = scr[_NG:_NG + _NG * batch]
        sems = scr[_NG + _NG * batch:]
        esem = sems[:_NG]
        xlsem = sems[_NG:_NG + _NG * batch]
        xssem = sems[_NG + _NG * batch:]
        wid = lax.axis_index("s") * info.num_cores + lax.axis_index("c")
        descs = {}

        def fire_loads(t):
            g = t % _NG
            lstart = (wid * lpw + t * _CROWS) * d
            descs[("e", t)] = pltpu.async_copy(
                emb_hbm.at[pl.ds(lstart, cw)], ebuf[g], esem[g])
            for b in range(batch):
                src = x_hbm.at[pl.ds((batch_off + b) * seq * d + lstart, cw)]
                descs[("x", t, b)] = pltpu.async_copy(
                    src, xbuf[g * batch + b], xlsem[g * batch + b])

        fire_loads(0)
        if n_chunks > 1:
            fire_loads(1)
        for t in range(n_chunks):
            g = t % _NG
            descs[("e", t)].wait()
            for b in range(batch):
                descs[("x", t, b)].wait()
            eb = ebuf[g]
            xbs = [xbuf[g * batch + b] for b in range(batch)]

            def body(i, _, eb=eb, xbs=xbs):
                for u in range(_UNROLL):
                    off = (i * _UNROLL + u) * 16
                    e = eb[pl.ds(off, 16)]
                    for xb in xbs:
                        plsc.addupdate(xb.at[pl.ds(off, 16)], e)
                return 0

            lstart = (wid * lpw + t * _CROWS) * d
            for b in range(batch):
                dst = out_hbm.at[pl.ds(b * seq * d + lstart, cw)]
                descs[("s", t, b)] = pltpu.async_copy(
                    xbuf[g * batch + b], dst, xssem[g * batch + b])
            if t + 2 < n_chunks:
                if t >= 1:
                    for b in range(batch):
                        descs[("s", t - 1, b)].wait()
                fire_loads(t + 2)
        for t in range(max(0, n_chunks - 3), n_chunks):
            for b in range(batch):
                if ("s", t, b) in descs:
                    descs[("s", t, b)].wait()

    return k


def _kernel_sc(x, emb_table):
    batch, seq, d = x.shape
    pos = emb_table[:seq]
    out = _make_sc(batch, seq, d)(x.reshape(-1), pos.reshape(-1))
    return out.reshape(batch, seq, d)


def _kernel_hybrid(x, emb_table, sc_batches=1):
    batch, seq, d = x.shape
    pos = emb_table[:seq]
    tc_b = batch - sc_batches
    tc_out = _kernel_tc(x, emb_table, nbatch=tc_b)
    sc_out = _make_sc(sc_batches, seq, d, batch_off=tc_b)(
        x.reshape(-1), pos.reshape(-1))
    return jnp.concatenate([tc_out, sc_out.reshape(sc_batches, seq, d)], axis=0)


def kernel(x, emb_table):
    return _kernel_sc(x, emb_table)


# speedup vs baseline: 5.2744x; 5.1401x over previous
import jax
import jax.numpy as jnp
from jax.experimental import pallas as pl


_BS = 2048


def _add_kernel(x_ref, emb_ref, out_ref):
    out_ref[...] = x_ref[...] + emb_ref[...]


def kernel(x, emb_table):
    batch, seq, d = x.shape
    pos = emb_table[:seq]
    bs = _BS if seq % _BS == 0 else seq
    grid = (seq // bs, batch)
    return pl.pallas_call(
        _add_kernel,
        grid=grid,
        in_specs=[
            pl.BlockSpec((1, bs, d), lambda i, j: (j, i, 0)),
            pl.BlockSpec((bs, d), lambda i, j: (i, 0)),
        ],
        out_specs=pl.BlockSpec((1, bs, d), lambda i, j: (j, i, 0)),
        out_shape=jax.ShapeDtypeStruct((batch, seq, d), x.dtype),
    )(x, pos)
